# baseline (speedup 1.0000x reference)
import functools

import jax
import jax.numpy as jnp
from jax import lax
from jax.experimental import pallas as pl
from jax.experimental.pallas import tpu as pltpu
from jax.experimental.pallas import tpu_sc as plsc

N = 10000
D = 128
K = 16
NPAD = 10240
RB = 256
GB = 400
E = 32
NW = 32
PER_W = NPAD // NW
CHUNKS = PER_W // E
TPR = NPAD // 16
ZC = TPR // E
DUMMY = N + 100
BIGI = 2 ** 30


def _topk_body(xr_ref, xt_ref, nn_ref, dist_ref):
    xr = xr_ref[...]
    xt = xt_ref[...]
    sqr = jnp.sum(xr * xr, axis=1, keepdims=True)
    sqc = jnp.sum(xt * xt, axis=0, keepdims=True)
    dist = sqr + sqc - 2.0 * jnp.dot(xr, xt, preferred_element_type=jnp.float32)
    ci = lax.broadcasted_iota(jnp.int32, (RB, NPAD), 1)
    dist_ref[...] = jnp.where(ci < N, dist, jnp.inf)
    li = lax.broadcasted_iota(jnp.int32, (RB, K), 1)

    def step(t, nn):
        d = dist_ref[...]
        m = jnp.min(d, axis=1, keepdims=True)
        idx = jnp.min(jnp.where(d == m, ci, BIGI), axis=1, keepdims=True)
        nn = jnp.where(li == t, idx, nn)
        dist_ref[...] = jnp.where(ci == idx, jnp.inf, d)
        return nn

    nn_ref[...] = lax.fori_loop(0, K, step, jnp.zeros((RB, K), jnp.int32))


def _topk(xpad, xt):
    return pl.pallas_call(
        _topk_body,
        grid=(NPAD // RB,),
        in_specs=[
            pl.BlockSpec((RB, D), lambda i: (i, 0)),
            pl.BlockSpec((D, NPAD), lambda i: (0, 0)),
        ],
        out_specs=pl.BlockSpec((RB, K), lambda i: (i, 0)),
        out_shape=jax.ShapeDtypeStruct((NPAD, K), jnp.int32),
        scratch_shapes=[pltpu.VMEM((RB, NPAD), jnp.float32)],
    )(xpad, xt)


def _mm_body(x_ref, t_ref, o_ref):
    o_ref[...] = jnp.dot(x_ref[...], t_ref[...], preferred_element_type=jnp.float32)


def _mm0(x, theta):
    return pl.pallas_call(
        _mm_body,
        grid=(N // GB,),
        in_specs=[
            pl.BlockSpec((GB, D), lambda i: (i, 0)),
            pl.BlockSpec((D, D), lambda i: (0, 0)),
        ],
        out_specs=pl.BlockSpec((GB, D), lambda i: (i, 0)),
        out_shape=jax.ShapeDtypeStruct((NPAD, D), jnp.float32),
    )(x, theta)


def _pre_act(p0, p1, d0, d1, b):
    dv = jnp.max(d0[0] + d1[0], axis=1, keepdims=True)
    pre = (p0[0] + p1[0]) * (jnp.float32(1.0 / 16.0) / dv)
    pre = pre + b[0:1, :]
    return jnp.where(pre >= 0, pre, jnp.float32(0.01) * pre)


def _layer_body(p0_ref, p1_ref, d0_ref, d1_ref, b_ref, t_ref, o_ref):
    h = _pre_act(p0_ref[...], p1_ref[...], d0_ref[...], d1_ref[...], b_ref[...])
    o_ref[...] = jnp.dot(h, t_ref[...], preferred_element_type=jnp.float32)


def _layer(parts, dparts, bias8, theta):
    return pl.pallas_call(
        _layer_body,
        grid=(N // GB,),
        in_specs=[
            pl.BlockSpec((1, GB, D), lambda i: (0, i, 0)),
            pl.BlockSpec((1, GB, D), lambda i: (1, i, 0)),
            pl.BlockSpec((1, GB, 16), lambda i: (0, i, 0)),
            pl.BlockSpec((1, GB, 16), lambda i: (1, i, 0)),
            pl.BlockSpec((8, D), lambda i: (0, 0)),
            pl.BlockSpec((D, D), lambda i: (0, 0)),
        ],
        out_specs=pl.BlockSpec((GB, D), lambda i: (i, 0)),
        out_shape=jax.ShapeDtypeStruct((NPAD, D), jnp.float32),
    )(parts, parts, dparts, dparts, bias8, theta)


def _final_body(p0_ref, p1_ref, d0_ref, d1_ref, b_ref, f_ref, s_ref):
    i = pl.program_id(0)
    h = _pre_act(p0_ref[...], p1_ref[...], d0_ref[...], d1_ref[...], b_ref[...])
    f_ref[...] = h

    @pl.when(i == 0)
    def _():
        s_ref[...] = jnp.zeros((8, D), jnp.float32)

    col = jnp.sum(h, axis=0, keepdims=True)
    s_ref[...] = s_ref[...] + jnp.broadcast_to(col, (8, D))


def _final(parts, dparts, bias8):
    return pl.pallas_call(
        _final_body,
        grid=(N // GB,),
        in_specs=[
            pl.BlockSpec((1, GB, D), lambda i: (0, i, 0)),
            pl.BlockSpec((1, GB, D), lambda i: (1, i, 0)),
            pl.BlockSpec((1, GB, 16), lambda i: (0, i, 0)),
            pl.BlockSpec((1, GB, 16), lambda i: (1, i, 0)),
            pl.BlockSpec((8, D), lambda i: (0, 0)),
        ],
        out_specs=[
            pl.BlockSpec((GB, D), lambda i: (i, 0)),
            pl.BlockSpec((8, D), lambda i: (0, 0)),
        ],
        out_shape=[
            jax.ShapeDtypeStruct((N, D), jnp.float32),
            jax.ShapeDtypeStruct((8, D), jnp.float32),
        ],
    )(parts, parts, dparts, dparts, bias8)


QN = NPAD // 4
TRASH = QN
QTPR = QN // 16
QZC = QTPR // E
DZC = TPR // E


def _sc_agg_body(xp_hbm, idx_hbm, xv_out,
                 idx_all, rows_v, xe_all, idx_adj, zero_v, xv_sh, sem,
                 sem2):
    c = lax.axis_index("c")
    s = lax.axis_index("s")
    w = c * 16 + s

    def initbuf(i, carry):
        for l in range(D // 16):
            zero_v[i, pl.ds(l * 16, 16)] = jnp.zeros((16,), jnp.float32)
        return carry

    lax.fori_loop(0, E, initbuf, 0)

    pltpu.sync_copy(idx_hbm.at[w], idx_all)
    sems = (sem, sem2)

    def chunk_gather(q, carry):
        def fire(g):
            return [
                pltpu.async_copy(xp_hbm.at[idx_all.at[q, 4 * g + m]],
                                 rows_v.at[g % 2, m], sems[g % 2])
                for m in range(4)
            ]

        handles = fire(0)
        for g in range(4):
            nxt = fire(g + 1) if g < 3 else []
            for h in handles:
                h.wait()
            buf = g % 2

            def rowsum(r, cc):
                for l in range(D // 16):
                    acc = rows_v[buf, 0, r, pl.ds(l * 16, 16)]
                    for m in range(1, 4):
                        acc = acc + rows_v[buf, m, r, pl.ds(l * 16, 16)]
                    if g == 0:
                        xe_all[q * E + r, pl.ds(l * 16, 16)] = acc
                    else:
                        xe_all[q * E + r, pl.ds(l * 16, 16)] = (
                            acc + xe_all[q * E + r, pl.ds(l * 16, 16)])
                return cc

            lax.fori_loop(0, E, rowsum, 0)
            handles = nxt
        return carry

    lax.fori_loop(0, CHUNKS, chunk_gather, 0)

    def quarter_body(quarter, carry0):
        base = quarter * QN

        zh = [
            pltpu.async_copy(zero_v, xv_sh.at[pl.ds(s * QTPR + kk * E, E), :],
                             sem)
            for kk in range(QZC)
        ]
        for h in zh:
            h.wait()
        plsc.subcore_barrier()

        def chunk_scatter(q, carry):
            for m in range(16):
                for cc2 in range(2):
                    for e2 in range(E // 16):
                        v = idx_all[2 * q + cc2, m, pl.ds(e2 * 16, 16)]
                        inr = (v >= base) & (v < base + QN)
                        idx_adj[m, pl.ds(cc2 * E + e2 * 16, 16)] = jnp.where(
                            inr, v - base, TRASH + m)
            handles = [
                pltpu.async_copy(xe_all.at[pl.ds(2 * q * E, 2 * E), :],
                                 xv_sh.at[idx_adj.at[m]], sem, add=True)
                for m in range(16)
            ]
            for h in handles:
                h.wait()
            return carry

        lax.fori_loop(0, CHUNKS // 2, chunk_scatter, 0)
        plsc.subcore_barrier()

        pltpu.sync_copy(xv_sh.at[pl.ds(s * QTPR, QTPR), :],
                        xv_out.at[c, pl.ds(base + s * QTPR, QTPR), :])
        return carry0

    lax.fori_loop(0, 4, quarter_body, 0)


@functools.lru_cache(maxsize=None)
def _sc_agg_kernel():
    return functools.partial(
        pl.kernel,
        out_type=jax.ShapeDtypeStruct((2, NPAD, D), jnp.float32),
        mesh=plsc.VectorSubcoreMesh(core_axis_name="c", subcore_axis_name="s",
                                    num_cores=2, num_subcores=16),
        scratch_types=[
            pltpu.VMEM((CHUNKS, 16, E), jnp.int32),
            pltpu.VMEM((2, 4, E, D), jnp.float32),
            pltpu.VMEM((PER_W, D), jnp.float32),
            pltpu.VMEM((16, 2 * E), jnp.int32),
            pltpu.VMEM((E, D), jnp.float32),
            pltpu.VMEM_SHARED((QN + E, D), jnp.float32),
            pltpu.SemaphoreType.DMA,
            pltpu.SemaphoreType.DMA,
        ],
    )(_sc_agg_body)


def _sc_degree_body(idx_hbm, dv_out, idx_all, ones_v, zerod_v, dv_sh):
    c = lax.axis_index("c")
    s = lax.axis_index("s")
    w = c * 16 + s

    def initbuf(i, carry):
        zerod_v[i, :] = jnp.zeros((16,), jnp.float32)
        ones_v[i, :] = jnp.ones((16,), jnp.float32)
        return carry

    lax.fori_loop(0, E, initbuf, 0)
    pltpu.sync_copy(idx_hbm.at[w], idx_all)

    def dzloop(kk, carry):
        pltpu.sync_copy(zerod_v, dv_sh.at[pl.ds(s * TPR + kk * E, E), :])
        return carry

    lax.fori_loop(0, DZC, dzloop, 0)
    plsc.subcore_barrier()

    def chunk_ones(q, carry):
        for m in range(16):
            pltpu.sync_copy(ones_v, dv_sh.at[idx_all.at[q, m]], add=True)
        return carry

    lax.fori_loop(0, CHUNKS, chunk_ones, 0)
    plsc.subcore_barrier()

    pltpu.sync_copy(dv_sh.at[pl.ds(s * TPR, TPR), :],
                    dv_out.at[c, pl.ds(s * TPR, TPR), :])


@functools.lru_cache(maxsize=None)
def _sc_degree_kernel():
    return functools.partial(
        pl.kernel,
        out_type=jax.ShapeDtypeStruct((2, NPAD, 16), jnp.float32),
        mesh=plsc.VectorSubcoreMesh(core_axis_name="c", subcore_axis_name="s",
                                    num_cores=2, num_subcores=16),
        scratch_types=[
            pltpu.VMEM((CHUNKS, 16, E), jnp.int32),
            pltpu.VMEM((E, 16), jnp.float32),
            pltpu.VMEM((E, 16), jnp.float32),
            pltpu.VMEM_SHARED((NPAD, 16), jnp.float32),
        ],
    )(_sc_degree_body)


def _sc_agg(xp, idxarr):
    return _sc_agg_kernel()(xp, idxarr)


def _sc_degree(idxarr):
    return _sc_degree_kernel()(idxarr)


def kernel(x, theta0, bias0, theta1, bias1, theta2, bias2, theta3, bias3,
           Wm, bm, Wa, ba):
    f32 = jnp.float32
    x = x.astype(f32)
    xpad = jnp.zeros((NPAD, D), f32).at[:N].set(x)
    nn_full = _topk(xpad, xpad.T)
    nnp = jnp.concatenate(
        [nn_full[:N], jnp.full((NPAD - N, K), DUMMY, jnp.int32)], axis=0)
    idxarr = nnp.T.reshape(16, NW, CHUNKS, E).transpose(1, 2, 0, 3)

    thetas = [theta1, theta2, theta3]
    biases = [bias0, bias1, bias2, bias3]

    dparts = _sc_degree(idxarr)
    h = _mm0(x, theta0)
    for L in range(3):
        parts = _sc_agg(h, idxarr)
        b8 = jnp.broadcast_to(biases[L][None, :], (8, D))
        h = _layer(parts, dparts, b8, thetas[L])
    parts = _sc_agg(h, idxarr)
    b8 = jnp.broadcast_to(biases[3][None, :], (8, D))
    feats, psum = _final(parts, dparts, b8)

    feats_pool = psum[0:1, :] * f32(1.0 / N)
    mean = (feats_pool @ Wm.T + bm)[0]
    alpha = (feats_pool @ Wa.T + ba)[0]
    return (jax.nn.sigmoid(mean), jnp.log(jax.nn.sigmoid(alpha)),
            feats, feats_pool)


# speedup vs baseline: 1.0221x; 1.0221x over previous
import functools

import jax
import jax.numpy as jnp
from jax import lax
from jax.experimental import pallas as pl
from jax.experimental.pallas import tpu as pltpu
from jax.experimental.pallas import tpu_sc as plsc

N = 10000
D = 128
K = 16
NPAD = 10240
RB = 256
GB = 400
E = 32
NW = 32
PER_W = NPAD // NW
CHUNKS = PER_W // E
TPR = NPAD // 16
ZC = TPR // E
DUMMY = N + 100
BIGI = 2 ** 30


def _topk_body(xr_ref, xt_ref, nn_ref, dist_ref):
    xr = xr_ref[...]
    xt = xt_ref[...]
    sqr = jnp.sum(xr * xr, axis=1, keepdims=True)
    sqc = jnp.sum(xt * xt, axis=0, keepdims=True)
    dist = sqr + sqc - 2.0 * jnp.dot(xr, xt, preferred_element_type=jnp.float32)
    ci = lax.broadcasted_iota(jnp.int32, (RB, NPAD), 1)
    dist_ref[...] = jnp.where(ci < N, dist, jnp.inf)
    li = lax.broadcasted_iota(jnp.int32, (RB, K), 1)

    def step(t, nn):
        d = dist_ref[...]
        m = jnp.min(d, axis=1, keepdims=True)
        idx = jnp.min(jnp.where(d == m, ci, BIGI), axis=1, keepdims=True)
        nn = jnp.where(li == t, idx, nn)
        dist_ref[...] = jnp.where(ci == idx, jnp.inf, d)
        return nn

    nn_ref[...] = lax.fori_loop(0, K, step, jnp.zeros((RB, K), jnp.int32))


def _topk(xpad, xt):
    return pl.pallas_call(
        _topk_body,
        grid=(NPAD // RB,),
        in_specs=[
            pl.BlockSpec((RB, D), lambda i: (i, 0)),
            pl.BlockSpec((D, NPAD), lambda i: (0, 0)),
        ],
        out_specs=pl.BlockSpec((RB, K), lambda i: (i, 0)),
        out_shape=jax.ShapeDtypeStruct((NPAD, K), jnp.int32),
        scratch_shapes=[pltpu.VMEM((RB, NPAD), jnp.float32)],
    )(xpad, xt)


def _mm_body(x_ref, t_ref, o_ref):
    o_ref[...] = jnp.dot(x_ref[...], t_ref[...], preferred_element_type=jnp.float32)


def _mm0(x, theta):
    return pl.pallas_call(
        _mm_body,
        grid=(N // GB,),
        in_specs=[
            pl.BlockSpec((GB, D), lambda i: (i, 0)),
            pl.BlockSpec((D, D), lambda i: (0, 0)),
        ],
        out_specs=pl.BlockSpec((GB, D), lambda i: (i, 0)),
        out_shape=jax.ShapeDtypeStruct((NPAD, D), jnp.float32),
    )(x, theta)


def _pre_act(p0, p1, d0, d1, b):
    dv = jnp.max(d0[0] + d1[0], axis=1, keepdims=True)
    pre = (p0[0] + p1[0]) * (jnp.float32(1.0 / 16.0) / dv)
    pre = pre + b[0:1, :]
    return jnp.where(pre >= 0, pre, jnp.float32(0.01) * pre)


def _layer_body(p0_ref, p1_ref, d0_ref, d1_ref, b_ref, t_ref, o_ref):
    h = _pre_act(p0_ref[...], p1_ref[...], d0_ref[...], d1_ref[...], b_ref[...])
    o_ref[...] = jnp.dot(h, t_ref[...], preferred_element_type=jnp.float32)


def _layer(parts, dparts, bias8, theta):
    return pl.pallas_call(
        _layer_body,
        grid=(N // GB,),
        in_specs=[
            pl.BlockSpec((1, GB, D), lambda i: (0, i, 0)),
            pl.BlockSpec((1, GB, D), lambda i: (1, i, 0)),
            pl.BlockSpec((1, GB, 16), lambda i: (0, i, 0)),
            pl.BlockSpec((1, GB, 16), lambda i: (1, i, 0)),
            pl.BlockSpec((8, D), lambda i: (0, 0)),
            pl.BlockSpec((D, D), lambda i: (0, 0)),
        ],
        out_specs=pl.BlockSpec((GB, D), lambda i: (i, 0)),
        out_shape=jax.ShapeDtypeStruct((NPAD, D), jnp.float32),
    )(parts, parts, dparts, dparts, bias8, theta)


def _final_body(p0_ref, p1_ref, d0_ref, d1_ref, b_ref, f_ref, s_ref):
    i = pl.program_id(0)
    h = _pre_act(p0_ref[...], p1_ref[...], d0_ref[...], d1_ref[...], b_ref[...])
    f_ref[...] = h

    @pl.when(i == 0)
    def _():
        s_ref[...] = jnp.zeros((8, D), jnp.float32)

    col = jnp.sum(h, axis=0, keepdims=True)
    s_ref[...] = s_ref[...] + jnp.broadcast_to(col, (8, D))


def _final(parts, dparts, bias8):
    return pl.pallas_call(
        _final_body,
        grid=(N // GB,),
        in_specs=[
            pl.BlockSpec((1, GB, D), lambda i: (0, i, 0)),
            pl.BlockSpec((1, GB, D), lambda i: (1, i, 0)),
            pl.BlockSpec((1, GB, 16), lambda i: (0, i, 0)),
            pl.BlockSpec((1, GB, 16), lambda i: (1, i, 0)),
            pl.BlockSpec((8, D), lambda i: (0, 0)),
        ],
        out_specs=[
            pl.BlockSpec((GB, D), lambda i: (i, 0)),
            pl.BlockSpec((8, D), lambda i: (0, 0)),
        ],
        out_shape=[
            jax.ShapeDtypeStruct((N, D), jnp.float32),
            jax.ShapeDtypeStruct((8, D), jnp.float32),
        ],
    )(parts, parts, dparts, dparts, bias8)


QN = NPAD // 4
TRASH = QN
QTPR = QN // 16
QZC = QTPR // E
DZC = TPR // E


def _sc_agg_body(xp_hbm, idx_hbm, xv_out,
                 idx_all, rows_v, xe_all, idx_adj, zero_v, xv_sh, sem,
                 sem2):
    c = lax.axis_index("c")
    s = lax.axis_index("s")
    w = c * 16 + s

    def initbuf(i, carry):
        for l in range(D // 16):
            zero_v[i, pl.ds(l * 16, 16)] = jnp.zeros((16,), jnp.float32)
        return carry

    lax.fori_loop(0, E, initbuf, 0)

    pltpu.sync_copy(idx_hbm.at[w], idx_all)
    sems = (sem, sem2)

    def chunk_gather(q, carry):
        def fire(g):
            return [
                pltpu.async_copy(xp_hbm.at[idx_all.at[q, 4 * g + m]],
                                 rows_v.at[g % 2, m], sems[g % 2])
                for m in range(4)
            ]

        handles = fire(0)
        for g in range(4):
            nxt = fire(g + 1) if g < 3 else []
            for h in handles:
                h.wait()
            buf = g % 2

            def rowsum(r, cc):
                for l in range(D // 16):
                    acc = rows_v[buf, 0, r, pl.ds(l * 16, 16)]
                    for m in range(1, 4):
                        acc = acc + rows_v[buf, m, r, pl.ds(l * 16, 16)]
                    if g == 0:
                        xe_all[q * E + r, pl.ds(l * 16, 16)] = acc
                    else:
                        xe_all[q * E + r, pl.ds(l * 16, 16)] = (
                            acc + xe_all[q * E + r, pl.ds(l * 16, 16)])
                return cc

            lax.fori_loop(0, E, rowsum, 0)
            handles = nxt
        return carry

    lax.fori_loop(0, CHUNKS, chunk_gather, 0)

    def quarter_body(quarter, carry0):
        base = quarter * QN

        zh = [
            pltpu.async_copy(zero_v, xv_sh.at[pl.ds(s * QTPR + kk * E, E), :],
                             sem)
            for kk in range(QZC)
        ]
        for h in zh:
            h.wait()
        plsc.subcore_barrier()

        def chunk_scatter(q, carry):
            for m in range(16):
                for e2 in range(E // 16):
                    v = idx_all[q, m, pl.ds(e2 * 16, 16)]
                    inr = (v >= base) & (v < base + QN)
                    idx_adj[m, pl.ds(e2 * 16, 16)] = jnp.where(
                        inr, v - base, TRASH + m)
            handles = [
                pltpu.async_copy(xe_all.at[pl.ds(q * E, E), :],
                                 xv_sh.at[idx_adj.at[m]], sem, add=True)
                for m in range(16)
            ]
            for h in handles:
                h.wait()
            return carry

        lax.fori_loop(0, CHUNKS, chunk_scatter, 0)
        plsc.subcore_barrier()

        pltpu.sync_copy(xv_sh.at[pl.ds(s * QTPR, QTPR), :],
                        xv_out.at[c, pl.ds(base + s * QTPR, QTPR), :])
        return carry0

    lax.fori_loop(0, 4, quarter_body, 0)


@functools.lru_cache(maxsize=None)
def _sc_agg_kernel():
    return functools.partial(
        pl.kernel,
        out_type=jax.ShapeDtypeStruct((2, NPAD, D), jnp.float32),
        mesh=plsc.VectorSubcoreMesh(core_axis_name="c", subcore_axis_name="s",
                                    num_cores=2, num_subcores=16),
        scratch_types=[
            pltpu.VMEM((CHUNKS, 16, E), jnp.int32),
            pltpu.VMEM((2, 4, E, D), jnp.float32),
            pltpu.VMEM((PER_W, D), jnp.float32),
            pltpu.VMEM((16, E), jnp.int32),
            pltpu.VMEM((E, D), jnp.float32),
            pltpu.VMEM_SHARED((QN + E, D), jnp.float32),
            pltpu.SemaphoreType.DMA,
            pltpu.SemaphoreType.DMA,
        ],
    )(_sc_agg_body)


def _sc_degree_body(idx_hbm, dv_out, idx_all, ones_v, zerod_v, dv_sh):
    c = lax.axis_index("c")
    s = lax.axis_index("s")
    w = c * 16 + s

    def initbuf(i, carry):
        zerod_v[i, :] = jnp.zeros((16,), jnp.float32)
        ones_v[i, :] = jnp.ones((16,), jnp.float32)
        return carry

    lax.fori_loop(0, E, initbuf, 0)
    pltpu.sync_copy(idx_hbm.at[w], idx_all)

    def dzloop(kk, carry):
        pltpu.sync_copy(zerod_v, dv_sh.at[pl.ds(s * TPR + kk * E, E), :])
        return carry

    lax.fori_loop(0, DZC, dzloop, 0)
    plsc.subcore_barrier()

    def chunk_ones(q, carry):
        for m in range(16):
            pltpu.sync_copy(ones_v, dv_sh.at[idx_all.at[q, m]], add=True)
        return carry

    lax.fori_loop(0, CHUNKS, chunk_ones, 0)
    plsc.subcore_barrier()

    pltpu.sync_copy(dv_sh.at[pl.ds(s * TPR, TPR), :],
                    dv_out.at[c, pl.ds(s * TPR, TPR), :])


@functools.lru_cache(maxsize=None)
def _sc_degree_kernel():
    return functools.partial(
        pl.kernel,
        out_type=jax.ShapeDtypeStruct((2, NPAD, 16), jnp.float32),
        mesh=plsc.VectorSubcoreMesh(core_axis_name="c", subcore_axis_name="s",
                                    num_cores=2, num_subcores=16),
        scratch_types=[
            pltpu.VMEM((CHUNKS, 16, E), jnp.int32),
            pltpu.VMEM((E, 16), jnp.float32),
            pltpu.VMEM((E, 16), jnp.float32),
            pltpu.VMEM_SHARED((NPAD, 16), jnp.float32),
        ],
    )(_sc_degree_body)


def _sc_agg(xp, idxarr):
    return _sc_agg_kernel()(xp, idxarr)


def _sc_degree(idxarr):
    return _sc_degree_kernel()(idxarr)


def kernel(x, theta0, bias0, theta1, bias1, theta2, bias2, theta3, bias3,
           Wm, bm, Wa, ba):
    f32 = jnp.float32
    x = x.astype(f32)
    xpad = jnp.zeros((NPAD, D), f32).at[:N].set(x)
    nn_full = _topk(xpad, xpad.T)
    nnp = jnp.concatenate(
        [nn_full[:N], jnp.full((NPAD - N, K), DUMMY, jnp.int32)], axis=0)
    idxarr = nnp.T.reshape(16, NW, CHUNKS, E).transpose(1, 2, 0, 3)

    thetas = [theta1, theta2, theta3]
    biases = [bias0, bias1, bias2, bias3]

    dparts = _sc_degree(idxarr)
    h = _mm0(x, theta0)
    for L in range(3):
        parts = _sc_agg(h, idxarr)
        b8 = jnp.broadcast_to(biases[L][None, :], (8, D))
        h = _layer(parts, dparts, b8, thetas[L])
    parts = _sc_agg(h, idxarr)
    b8 = jnp.broadcast_to(biases[3][None, :], (8, D))
    feats, psum = _final(parts, dparts, b8)

    feats_pool = psum[0:1, :] * f32(1.0 / N)
    mean = (feats_pool @ Wm.T + bm)[0]
    alpha = (feats_pool @ Wa.T + ba)[0]
    return (jax.nn.sigmoid(mean), jnp.log(jax.nn.sigmoid(alpha)),
            feats, feats_pool)


# speedup vs baseline: 1.0354x; 1.0131x over previous
import functools

import jax
import jax.numpy as jnp
from jax import lax
from jax.experimental import pallas as pl
from jax.experimental.pallas import tpu as pltpu
from jax.experimental.pallas import tpu_sc as plsc

N = 10000
D = 128
K = 16
NPAD = 10240
RB = 512
GB = 400
E = 32
NW = 32
PER_W = NPAD // NW
CHUNKS = PER_W // E
TPR = NPAD // 16
ZC = TPR // E
DUMMY = N + 100
BIGI = 2 ** 30


def _topk_body(xr_ref, xt_ref, nn_ref, dist_ref):
    xr = xr_ref[...]
    xt = xt_ref[...]
    sqr = jnp.sum(xr * xr, axis=1, keepdims=True)
    sqc = jnp.sum(xt * xt, axis=0, keepdims=True)
    dist = sqr + sqc - 2.0 * jnp.dot(xr, xt, preferred_element_type=jnp.float32)
    ci = lax.broadcasted_iota(jnp.int32, (RB, NPAD), 1)
    dist_ref[...] = jnp.where(ci < N, dist, jnp.inf)
    li = lax.broadcasted_iota(jnp.int32, (RB, K), 1)

    def step(t, nn):
        d = dist_ref[...]
        m = jnp.min(d, axis=1, keepdims=True)
        idx = jnp.min(jnp.where(d == m, ci, BIGI), axis=1, keepdims=True)
        nn = jnp.where(li == t, idx, nn)
        dist_ref[...] = jnp.where(ci == idx, jnp.inf, d)
        return nn

    nn_ref[...] = lax.fori_loop(0, K, step, jnp.zeros((RB, K), jnp.int32))


def _topk(xpad, xt):
    return pl.pallas_call(
        _topk_body,
        grid=(NPAD // RB,),
        in_specs=[
            pl.BlockSpec((RB, D), lambda i: (i, 0)),
            pl.BlockSpec((D, NPAD), lambda i: (0, 0)),
        ],
        out_specs=pl.BlockSpec((RB, K), lambda i: (i, 0)),
        out_shape=jax.ShapeDtypeStruct((NPAD, K), jnp.int32),
        scratch_shapes=[pltpu.VMEM((RB, NPAD), jnp.float32)],
    )(xpad, xt)


def _mm_body(x_ref, t_ref, o_ref):
    o_ref[...] = jnp.dot(x_ref[...], t_ref[...], preferred_element_type=jnp.float32)


def _mm0(x, theta):
    return pl.pallas_call(
        _mm_body,
        grid=(N // GB,),
        in_specs=[
            pl.BlockSpec((GB, D), lambda i: (i, 0)),
            pl.BlockSpec((D, D), lambda i: (0, 0)),
        ],
        out_specs=pl.BlockSpec((GB, D), lambda i: (i, 0)),
        out_shape=jax.ShapeDtypeStruct((NPAD, D), jnp.float32),
    )(x, theta)


def _pre_act(p0, p1, d0, d1, b):
    dv = jnp.max(d0[0] + d1[0], axis=1, keepdims=True)
    pre = (p0[0] + p1[0]) * (jnp.float32(1.0 / 16.0) / dv)
    pre = pre + b[0:1, :]
    return jnp.where(pre >= 0, pre, jnp.float32(0.01) * pre)


def _layer_body(p0_ref, p1_ref, d0_ref, d1_ref, b_ref, t_ref, o_ref):
    h = _pre_act(p0_ref[...], p1_ref[...], d0_ref[...], d1_ref[...], b_ref[...])
    o_ref[...] = jnp.dot(h, t_ref[...], preferred_element_type=jnp.float32)


def _layer(parts, dparts, bias8, theta):
    return pl.pallas_call(
        _layer_body,
        grid=(N // GB,),
        in_specs=[
            pl.BlockSpec((1, GB, D), lambda i: (0, i, 0)),
            pl.BlockSpec((1, GB, D), lambda i: (1, i, 0)),
            pl.BlockSpec((1, GB, 16), lambda i: (0, i, 0)),
            pl.BlockSpec((1, GB, 16), lambda i: (1, i, 0)),
            pl.BlockSpec((8, D), lambda i: (0, 0)),
            pl.BlockSpec((D, D), lambda i: (0, 0)),
        ],
        out_specs=pl.BlockSpec((GB, D), lambda i: (i, 0)),
        out_shape=jax.ShapeDtypeStruct((NPAD, D), jnp.float32),
    )(parts, parts, dparts, dparts, bias8, theta)


def _final_body(p0_ref, p1_ref, d0_ref, d1_ref, b_ref, f_ref, s_ref):
    i = pl.program_id(0)
    h = _pre_act(p0_ref[...], p1_ref[...], d0_ref[...], d1_ref[...], b_ref[...])
    f_ref[...] = h

    @pl.when(i == 0)
    def _():
        s_ref[...] = jnp.zeros((8, D), jnp.float32)

    col = jnp.sum(h, axis=0, keepdims=True)
    s_ref[...] = s_ref[...] + jnp.broadcast_to(col, (8, D))


def _final(parts, dparts, bias8):
    return pl.pallas_call(
        _final_body,
        grid=(N // GB,),
        in_specs=[
            pl.BlockSpec((1, GB, D), lambda i: (0, i, 0)),
            pl.BlockSpec((1, GB, D), lambda i: (1, i, 0)),
            pl.BlockSpec((1, GB, 16), lambda i: (0, i, 0)),
            pl.BlockSpec((1, GB, 16), lambda i: (1, i, 0)),
            pl.BlockSpec((8, D), lambda i: (0, 0)),
        ],
        out_specs=[
            pl.BlockSpec((GB, D), lambda i: (i, 0)),
            pl.BlockSpec((8, D), lambda i: (0, 0)),
        ],
        out_shape=[
            jax.ShapeDtypeStruct((N, D), jnp.float32),
            jax.ShapeDtypeStruct((8, D), jnp.float32),
        ],
    )(parts, parts, dparts, dparts, bias8)


QN = NPAD // 4
TRASH = QN
QTPR = QN // 16
QZC = QTPR // E
DZC = TPR // E


def _sc_agg_body(xp_hbm, idx_hbm, xv_out,
                 idx_all, rows_v, xe_all, idx_adj, zero_v, xv_sh, sem,
                 sem2):
    c = lax.axis_index("c")
    s = lax.axis_index("s")
    w = c * 16 + s

    def initbuf(i, carry):
        for l in range(D // 16):
            zero_v[i, pl.ds(l * 16, 16)] = jnp.zeros((16,), jnp.float32)
        return carry

    lax.fori_loop(0, E, initbuf, 0)

    pltpu.sync_copy(idx_hbm.at[w], idx_all)
    sems = (sem, sem2)

    def chunk_gather(q, carry):
        def fire(g):
            return [
                pltpu.async_copy(xp_hbm.at[idx_all.at[q, 4 * g + m]],
                                 rows_v.at[g % 2, m], sems[g % 2])
                for m in range(4)
            ]

        handles = fire(0)
        for g in range(4):
            nxt = fire(g + 1) if g < 3 else []
            for h in handles:
                h.wait()
            buf = g % 2

            def rowsum(r, cc):
                for l in range(D // 16):
                    acc = rows_v[buf, 0, r, pl.ds(l * 16, 16)]
                    for m in range(1, 4):
                        acc = acc + rows_v[buf, m, r, pl.ds(l * 16, 16)]
                    if g == 0:
                        xe_all[q * E + r, pl.ds(l * 16, 16)] = acc
                    else:
                        xe_all[q * E + r, pl.ds(l * 16, 16)] = (
                            acc + xe_all[q * E + r, pl.ds(l * 16, 16)])
                return cc

            lax.fori_loop(0, E, rowsum, 0)
            handles = nxt
        return carry

    lax.fori_loop(0, CHUNKS, chunk_gather, 0)

    def quarter_body(quarter, carry0):
        base = quarter * QN

        zh = [
            pltpu.async_copy(zero_v, xv_sh.at[pl.ds(s * QTPR + kk * E, E), :],
                             sem)
            for kk in range(QZC)
        ]
        for h in zh:
            h.wait()
        plsc.subcore_barrier()

        def chunk_scatter(q, carry):
            for m in range(16):
                for e2 in range(E // 16):
                    v = idx_all[q, m, pl.ds(e2 * 16, 16)]
                    inr = (v >= base) & (v < base + QN)
                    idx_adj[m, pl.ds(e2 * 16, 16)] = jnp.where(
                        inr, v - base, TRASH + m)
            handles = [
                pltpu.async_copy(xe_all.at[pl.ds(q * E, E), :],
                                 xv_sh.at[idx_adj.at[m]], sem, add=True)
                for m in range(16)
            ]
            for h in handles:
                h.wait()
            return carry

        lax.fori_loop(0, CHUNKS, chunk_scatter, 0)
        plsc.subcore_barrier()

        pltpu.sync_copy(xv_sh.at[pl.ds(s * QTPR, QTPR), :],
                        xv_out.at[c, pl.ds(base + s * QTPR, QTPR), :])
        return carry0

    lax.fori_loop(0, 4, quarter_body, 0)


@functools.lru_cache(maxsize=None)
def _sc_agg_kernel():
    return functools.partial(
        pl.kernel,
        out_type=jax.ShapeDtypeStruct((2, NPAD, D), jnp.float32),
        mesh=plsc.VectorSubcoreMesh(core_axis_name="c", subcore_axis_name="s",
                                    num_cores=2, num_subcores=16),
        scratch_types=[
            pltpu.VMEM((CHUNKS, 16, E), jnp.int32),
            pltpu.VMEM((2, 4, E, D), jnp.float32),
            pltpu.VMEM((PER_W, D), jnp.float32),
            pltpu.VMEM((16, E), jnp.int32),
            pltpu.VMEM((E, D), jnp.float32),
            pltpu.VMEM_SHARED((QN + E, D), jnp.float32),
            pltpu.SemaphoreType.DMA,
            pltpu.SemaphoreType.DMA,
        ],
    )(_sc_agg_body)


def _sc_degree_body(idx_hbm, dv_out, idx_all, ones_v, zerod_v, dv_sh):
    c = lax.axis_index("c")
    s = lax.axis_index("s")
    w = c * 16 + s

    def initbuf(i, carry):
        zerod_v[i, :] = jnp.zeros((16,), jnp.float32)
        ones_v[i, :] = jnp.ones((16,), jnp.float32)
        return carry

    lax.fori_loop(0, E, initbuf, 0)
    pltpu.sync_copy(idx_hbm.at[w], idx_all)

    def dzloop(kk, carry):
        pltpu.sync_copy(zerod_v, dv_sh.at[pl.ds(s * TPR + kk * E, E), :])
        return carry

    lax.fori_loop(0, DZC, dzloop, 0)
    plsc.subcore_barrier()

    def chunk_ones(q, carry):
        for m in range(16):
            pltpu.sync_copy(ones_v, dv_sh.at[idx_all.at[q, m]], add=True)
        return carry

    lax.fori_loop(0, CHUNKS, chunk_ones, 0)
    plsc.subcore_barrier()

    pltpu.sync_copy(dv_sh.at[pl.ds(s * TPR, TPR), :],
                    dv_out.at[c, pl.ds(s * TPR, TPR), :])


@functools.lru_cache(maxsize=None)
def _sc_degree_kernel():
    return functools.partial(
        pl.kernel,
        out_type=jax.ShapeDtypeStruct((2, NPAD, 16), jnp.float32),
        mesh=plsc.VectorSubcoreMesh(core_axis_name="c", subcore_axis_name="s",
                                    num_cores=2, num_subcores=16),
        scratch_types=[
            pltpu.VMEM((CHUNKS, 16, E), jnp.int32),
            pltpu.VMEM((E, 16), jnp.float32),
            pltpu.VMEM((E, 16), jnp.float32),
            pltpu.VMEM_SHARED((NPAD, 16), jnp.float32),
        ],
    )(_sc_degree_body)


def _sc_agg(xp, idxarr):
    return _sc_agg_kernel()(xp, idxarr)


def _sc_degree(idxarr):
    return _sc_degree_kernel()(idxarr)


def kernel(x, theta0, bias0, theta1, bias1, theta2, bias2, theta3, bias3,
           Wm, bm, Wa, ba):
    f32 = jnp.float32
    x = x.astype(f32)
    xpad = jnp.zeros((NPAD, D), f32).at[:N].set(x)
    nn_full = _topk(xpad, xpad.T)
    nnp = jnp.concatenate(
        [nn_full[:N], jnp.full((NPAD - N, K), DUMMY, jnp.int32)], axis=0)
    idxarr = nnp.T.reshape(16, NW, CHUNKS, E).transpose(1, 2, 0, 3)

    thetas = [theta1, theta2, theta3]
    biases = [bias0, bias1, bias2, bias3]

    dparts = _sc_degree(idxarr)
    h = _mm0(x, theta0)
    for L in range(3):
        parts = _sc_agg(h, idxarr)
        b8 = jnp.broadcast_to(biases[L][None, :], (8, D))
        h = _layer(parts, dparts, b8, thetas[L])
    parts = _sc_agg(h, idxarr)
    b8 = jnp.broadcast_to(biases[3][None, :], (8, D))
    feats, psum = _final(parts, dparts, b8)

    feats_pool = psum[0:1, :] * f32(1.0 / N)
    mean = (feats_pool @ Wm.T + bm)[0]
    alpha = (feats_pool @ Wa.T + ba)[0]
    return (jax.nn.sigmoid(mean), jnp.log(jax.nn.sigmoid(alpha)),
            feats, feats_pool)


# speedup vs baseline: 1.0470x; 1.0112x over previous
import functools

import jax
import jax.numpy as jnp
from jax import lax
from jax.experimental import pallas as pl
from jax.experimental.pallas import tpu as pltpu
from jax.experimental.pallas import tpu_sc as plsc

N = 10000
D = 128
K = 16
NPAD = 10240
RB = 1024
GB = 400
E = 32
NW = 32
PER_W = NPAD // NW
CHUNKS = PER_W // E
TPR = NPAD // 16
ZC = TPR // E
DUMMY = N + 100
BIGI = 2 ** 30


def _topk_body(xr_ref, xt_ref, nn_ref, dist_ref):
    xr = xr_ref[...]
    xt = xt_ref[...]
    sqr = jnp.sum(xr * xr, axis=1, keepdims=True)
    sqc = jnp.sum(xt * xt, axis=0, keepdims=True)
    dist = sqr + sqc - 2.0 * jnp.dot(xr, xt, preferred_element_type=jnp.float32)
    ci = lax.broadcasted_iota(jnp.int32, (RB, NPAD), 1)
    dist_ref[...] = jnp.where(ci < N, dist, jnp.inf)
    li = lax.broadcasted_iota(jnp.int32, (RB, K), 1)

    def step(t, nn):
        d = dist_ref[...]
        m = jnp.min(d, axis=1, keepdims=True)
        idx = jnp.min(jnp.where(d == m, ci, BIGI), axis=1, keepdims=True)
        nn = jnp.where(li == t, idx, nn)
        dist_ref[...] = jnp.where(ci == idx, jnp.inf, d)
        return nn

    nn_ref[...] = lax.fori_loop(0, K, step, jnp.zeros((RB, K), jnp.int32))


def _topk(xpad, xt):
    return pl.pallas_call(
        _topk_body,
        grid=(NPAD // RB,),
        in_specs=[
            pl.BlockSpec((RB, D), lambda i: (i, 0)),
            pl.BlockSpec((D, NPAD), lambda i: (0, 0)),
        ],
        out_specs=pl.BlockSpec((RB, K), lambda i: (i, 0)),
        out_shape=jax.ShapeDtypeStruct((NPAD, K), jnp.int32),
        scratch_shapes=[pltpu.VMEM((RB, NPAD), jnp.float32)],
    )(xpad, xt)


def _mm_body(x_ref, t_ref, o_ref):
    o_ref[...] = jnp.dot(x_ref[...], t_ref[...], preferred_element_type=jnp.float32)


def _mm0(x, theta):
    return pl.pallas_call(
        _mm_body,
        grid=(N // GB,),
        in_specs=[
            pl.BlockSpec((GB, D), lambda i: (i, 0)),
            pl.BlockSpec((D, D), lambda i: (0, 0)),
        ],
        out_specs=pl.BlockSpec((GB, D), lambda i: (i, 0)),
        out_shape=jax.ShapeDtypeStruct((NPAD, D), jnp.float32),
    )(x, theta)


def _pre_act(p0, p1, d0, d1, b):
    dv = jnp.max(d0[0] + d1[0], axis=1, keepdims=True)
    pre = (p0[0] + p1[0]) * (jnp.float32(1.0 / 16.0) / dv)
    pre = pre + b[0:1, :]
    return jnp.where(pre >= 0, pre, jnp.float32(0.01) * pre)


def _layer_body(p0_ref, p1_ref, d0_ref, d1_ref, b_ref, t_ref, o_ref):
    h = _pre_act(p0_ref[...], p1_ref[...], d0_ref[...], d1_ref[...], b_ref[...])
    o_ref[...] = jnp.dot(h, t_ref[...], preferred_element_type=jnp.float32)


def _layer(parts, dparts, bias8, theta):
    return pl.pallas_call(
        _layer_body,
        grid=(N // GB,),
        in_specs=[
            pl.BlockSpec((1, GB, D), lambda i: (0, i, 0)),
            pl.BlockSpec((1, GB, D), lambda i: (1, i, 0)),
            pl.BlockSpec((1, GB, 16), lambda i: (0, i, 0)),
            pl.BlockSpec((1, GB, 16), lambda i: (1, i, 0)),
            pl.BlockSpec((8, D), lambda i: (0, 0)),
            pl.BlockSpec((D, D), lambda i: (0, 0)),
        ],
        out_specs=pl.BlockSpec((GB, D), lambda i: (i, 0)),
        out_shape=jax.ShapeDtypeStruct((NPAD, D), jnp.float32),
    )(parts, parts, dparts, dparts, bias8, theta)


def _final_body(p0_ref, p1_ref, d0_ref, d1_ref, b_ref, f_ref, s_ref):
    i = pl.program_id(0)
    h = _pre_act(p0_ref[...], p1_ref[...], d0_ref[...], d1_ref[...], b_ref[...])
    f_ref[...] = h

    @pl.when(i == 0)
    def _():
        s_ref[...] = jnp.zeros((8, D), jnp.float32)

    col = jnp.sum(h, axis=0, keepdims=True)
    s_ref[...] = s_ref[...] + jnp.broadcast_to(col, (8, D))


def _final(parts, dparts, bias8):
    return pl.pallas_call(
        _final_body,
        grid=(N // GB,),
        in_specs=[
            pl.BlockSpec((1, GB, D), lambda i: (0, i, 0)),
            pl.BlockSpec((1, GB, D), lambda i: (1, i, 0)),
            pl.BlockSpec((1, GB, 16), lambda i: (0, i, 0)),
            pl.BlockSpec((1, GB, 16), lambda i: (1, i, 0)),
            pl.BlockSpec((8, D), lambda i: (0, 0)),
        ],
        out_specs=[
            pl.BlockSpec((GB, D), lambda i: (i, 0)),
            pl.BlockSpec((8, D), lambda i: (0, 0)),
        ],
        out_shape=[
            jax.ShapeDtypeStruct((N, D), jnp.float32),
            jax.ShapeDtypeStruct((8, D), jnp.float32),
        ],
    )(parts, parts, dparts, dparts, bias8)


QN = NPAD // 4
TRASH = QN
QTPR = QN // 16
QZC = QTPR // E
DZC = TPR // E


def _sc_agg_body(xp_hbm, idx_hbm, xv_out,
                 idx_all, rows_v, xe_all, idx_adj, zero_v, xv_sh, sem,
                 sem2):
    c = lax.axis_index("c")
    s = lax.axis_index("s")
    w = c * 16 + s

    def initbuf(i, carry):
        for l in range(D // 16):
            zero_v[i, pl.ds(l * 16, 16)] = jnp.zeros((16,), jnp.float32)
        return carry

    lax.fori_loop(0, E, initbuf, 0)

    pltpu.sync_copy(idx_hbm.at[w], idx_all)
    sems = (sem, sem2)

    def chunk_gather(q, carry):
        def fire(g):
            return [
                pltpu.async_copy(xp_hbm.at[idx_all.at[q, 4 * g + m]],
                                 rows_v.at[g % 2, m], sems[g % 2])
                for m in range(4)
            ]

        handles = fire(0)
        for g in range(4):
            nxt = fire(g + 1) if g < 3 else []
            for h in handles:
                h.wait()
            buf = g % 2

            def rowsum(r, cc):
                for l in range(D // 16):
                    acc = rows_v[buf, 0, r, pl.ds(l * 16, 16)]
                    for m in range(1, 4):
                        acc = acc + rows_v[buf, m, r, pl.ds(l * 16, 16)]
                    if g == 0:
                        xe_all[q * E + r, pl.ds(l * 16, 16)] = acc
                    else:
                        xe_all[q * E + r, pl.ds(l * 16, 16)] = (
                            acc + xe_all[q * E + r, pl.ds(l * 16, 16)])
                return cc

            lax.fori_loop(0, E, rowsum, 0)
            handles = nxt
        return carry

    lax.fori_loop(0, CHUNKS, chunk_gather, 0)

    def quarter_body(quarter, carry0):
        base = quarter * QN

        zh = [
            pltpu.async_copy(zero_v, xv_sh.at[pl.ds(s * QTPR + kk * E, E), :],
                             sem)
            for kk in range(QZC)
        ]
        for h in zh:
            h.wait()
        plsc.subcore_barrier()

        def chunk_scatter(q, carry):
            for m in range(16):
                for e2 in range(E // 16):
                    v = idx_all[q, m, pl.ds(e2 * 16, 16)]
                    inr = (v >= base) & (v < base + QN)
                    idx_adj[m, pl.ds(e2 * 16, 16)] = jnp.where(
                        inr, v - base, TRASH + m)
            handles = [
                pltpu.async_copy(xe_all.at[pl.ds(q * E, E), :],
                                 xv_sh.at[idx_adj.at[m]], sem, add=True)
                for m in range(16)
            ]
            for h in handles:
                h.wait()
            return carry

        lax.fori_loop(0, CHUNKS, chunk_scatter, 0)
        plsc.subcore_barrier()

        pltpu.sync_copy(xv_sh.at[pl.ds(s * QTPR, QTPR), :],
                        xv_out.at[c, pl.ds(base + s * QTPR, QTPR), :])
        return carry0

    lax.fori_loop(0, 4, quarter_body, 0)


@functools.lru_cache(maxsize=None)
def _sc_agg_kernel():
    return functools.partial(
        pl.kernel,
        out_type=jax.ShapeDtypeStruct((2, NPAD, D), jnp.float32),
        mesh=plsc.VectorSubcoreMesh(core_axis_name="c", subcore_axis_name="s",
                                    num_cores=2, num_subcores=16),
        scratch_types=[
            pltpu.VMEM((CHUNKS, 16, E), jnp.int32),
            pltpu.VMEM((2, 4, E, D), jnp.float32),
            pltpu.VMEM((PER_W, D), jnp.float32),
            pltpu.VMEM((16, E), jnp.int32),
            pltpu.VMEM((E, D), jnp.float32),
            pltpu.VMEM_SHARED((QN + E, D), jnp.float32),
            pltpu.SemaphoreType.DMA,
            pltpu.SemaphoreType.DMA,
        ],
    )(_sc_agg_body)


def _sc_degree_body(idx_hbm, dv_out, idx_all, ones_v, zerod_v, dv_sh):
    c = lax.axis_index("c")
    s = lax.axis_index("s")
    w = c * 16 + s

    def initbuf(i, carry):
        zerod_v[i, :] = jnp.zeros((16,), jnp.float32)
        ones_v[i, :] = jnp.ones((16,), jnp.float32)
        return carry

    lax.fori_loop(0, E, initbuf, 0)
    pltpu.sync_copy(idx_hbm.at[w], idx_all)

    def dzloop(kk, carry):
        pltpu.sync_copy(zerod_v, dv_sh.at[pl.ds(s * TPR + kk * E, E), :])
        return carry

    lax.fori_loop(0, DZC, dzloop, 0)
    plsc.subcore_barrier()

    def chunk_ones(q, carry):
        for m in range(16):
            pltpu.sync_copy(ones_v, dv_sh.at[idx_all.at[q, m]], add=True)
        return carry

    lax.fori_loop(0, CHUNKS, chunk_ones, 0)
    plsc.subcore_barrier()

    pltpu.sync_copy(dv_sh.at[pl.ds(s * TPR, TPR), :],
                    dv_out.at[c, pl.ds(s * TPR, TPR), :])


@functools.lru_cache(maxsize=None)
def _sc_degree_kernel():
    return functools.partial(
        pl.kernel,
        out_type=jax.ShapeDtypeStruct((2, NPAD, 16), jnp.float32),
        mesh=plsc.VectorSubcoreMesh(core_axis_name="c", subcore_axis_name="s",
                                    num_cores=2, num_subcores=16),
        scratch_types=[
            pltpu.VMEM((CHUNKS, 16, E), jnp.int32),
            pltpu.VMEM((E, 16), jnp.float32),
            pltpu.VMEM((E, 16), jnp.float32),
            pltpu.VMEM_SHARED((NPAD, 16), jnp.float32),
        ],
    )(_sc_degree_body)


def _sc_agg(xp, idxarr):
    return _sc_agg_kernel()(xp, idxarr)


def _sc_degree(idxarr):
    return _sc_degree_kernel()(idxarr)


def kernel(x, theta0, bias0, theta1, bias1, theta2, bias2, theta3, bias3,
           Wm, bm, Wa, ba):
    f32 = jnp.float32
    x = x.astype(f32)
    xpad = jnp.zeros((NPAD, D), f32).at[:N].set(x)
    nn_full = _topk(xpad, xpad.T)
    nnp = jnp.concatenate(
        [nn_full[:N], jnp.full((NPAD - N, K), DUMMY, jnp.int32)], axis=0)
    idxarr = nnp.T.reshape(16, NW, CHUNKS, E).transpose(1, 2, 0, 3)

    thetas = [theta1, theta2, theta3]
    biases = [bias0, bias1, bias2, bias3]

    dparts = _sc_degree(idxarr)
    h = _mm0(x, theta0)
    for L in range(3):
        parts = _sc_agg(h, idxarr)
        b8 = jnp.broadcast_to(biases[L][None, :], (8, D))
        h = _layer(parts, dparts, b8, thetas[L])
    parts = _sc_agg(h, idxarr)
    b8 = jnp.broadcast_to(biases[3][None, :], (8, D))
    feats, psum = _final(parts, dparts, b8)

    feats_pool = psum[0:1, :] * f32(1.0 / N)
    mean = (feats_pool @ Wm.T + bm)[0]
    alpha = (feats_pool @ Wa.T + ba)[0]
    return (jax.nn.sigmoid(mean), jnp.log(jax.nn.sigmoid(alpha)),
            feats, feats_pool)


# speedup vs baseline: 1.0848x; 1.0361x over previous
import functools

import jax
import jax.numpy as jnp
from jax import lax
from jax.experimental import pallas as pl
from jax.experimental.pallas import tpu as pltpu
from jax.experimental.pallas import tpu_sc as plsc

N = 10000
D = 128
K = 16
NPAD = 10240
RB = 1024
GB = 400
E = 32
NW = 32
PER_W = NPAD // NW
CHUNKS = PER_W // E
TPR = NPAD // 16
ZC = TPR // E
DUMMY = N + 100
BIGI = 2 ** 30


def _topk_body(xr_ref, xt_ref, nn_ref, dist_ref):
    xr = xr_ref[...]
    xt = xt_ref[...]
    sqr = jnp.sum(xr * xr, axis=1, keepdims=True)
    sqc = jnp.sum(xt * xt, axis=0, keepdims=True)
    dist = sqr + sqc - 2.0 * jnp.dot(xr, xt, preferred_element_type=jnp.float32)
    ci = lax.broadcasted_iota(jnp.int32, (RB, NPAD), 1)
    dist_ref[...] = jnp.where(ci < N, dist, jnp.inf)
    li = lax.broadcasted_iota(jnp.int32, (RB, K), 1)

    def step(t, nn):
        d = dist_ref[...]
        m = jnp.min(d, axis=1, keepdims=True)
        idx = jnp.min(jnp.where(d == m, ci, BIGI), axis=1, keepdims=True)
        nn = jnp.where(li == t, idx, nn)
        dist_ref[...] = jnp.where(ci == idx, jnp.inf, d)
        return nn

    nn_ref[...] = lax.fori_loop(0, K, step, jnp.zeros((RB, K), jnp.int32))


def _topk(xpad, xt):
    return pl.pallas_call(
        _topk_body,
        grid=(NPAD // RB,),
        in_specs=[
            pl.BlockSpec((RB, D), lambda i: (i, 0)),
            pl.BlockSpec((D, NPAD), lambda i: (0, 0)),
        ],
        out_specs=pl.BlockSpec((RB, K), lambda i: (i, 0)),
        out_shape=jax.ShapeDtypeStruct((NPAD, K), jnp.int32),
        scratch_shapes=[pltpu.VMEM((RB, NPAD), jnp.float32)],
    )(xpad, xt)


def _mm_body(x_ref, t_ref, o_ref):
    o_ref[...] = jnp.dot(x_ref[...], t_ref[...], preferred_element_type=jnp.float32)


def _mm0(x, theta):
    return pl.pallas_call(
        _mm_body,
        grid=(N // GB,),
        in_specs=[
            pl.BlockSpec((GB, D), lambda i: (i, 0)),
            pl.BlockSpec((D, D), lambda i: (0, 0)),
        ],
        out_specs=pl.BlockSpec((GB, D), lambda i: (i, 0)),
        out_shape=jax.ShapeDtypeStruct((NPAD, D), jnp.float32),
    )(x, theta)


def _pre_act(p0, p1, d0, d1, b):
    dv = jnp.max(d0[0] + d1[0], axis=1, keepdims=True)
    pre = (p0[0] + p1[0]) * (jnp.float32(1.0 / 16.0) / dv)
    pre = pre + b[0:1, :]
    return jnp.where(pre >= 0, pre, jnp.float32(0.01) * pre)


def _layer_body(p0_ref, p1_ref, d0_ref, d1_ref, b_ref, t_ref, o_ref):
    h = _pre_act(p0_ref[...], p1_ref[...], d0_ref[...], d1_ref[...], b_ref[...])
    o_ref[...] = jnp.dot(h, t_ref[...], preferred_element_type=jnp.float32)


def _layer(parts, dparts, bias8, theta):
    return pl.pallas_call(
        _layer_body,
        grid=(N // GB,),
        in_specs=[
            pl.BlockSpec((1, GB, D), lambda i: (0, i, 0)),
            pl.BlockSpec((1, GB, D), lambda i: (1, i, 0)),
            pl.BlockSpec((1, GB, 16), lambda i: (0, i, 0)),
            pl.BlockSpec((1, GB, 16), lambda i: (1, i, 0)),
            pl.BlockSpec((8, D), lambda i: (0, 0)),
            pl.BlockSpec((D, D), lambda i: (0, 0)),
        ],
        out_specs=pl.BlockSpec((GB, D), lambda i: (i, 0)),
        out_shape=jax.ShapeDtypeStruct((NPAD, D), jnp.float32),
    )(parts, parts, dparts, dparts, bias8, theta)


def _final_body(p0_ref, p1_ref, d0_ref, d1_ref, b_ref, f_ref, s_ref):
    i = pl.program_id(0)
    h = _pre_act(p0_ref[...], p1_ref[...], d0_ref[...], d1_ref[...], b_ref[...])
    f_ref[...] = h

    @pl.when(i == 0)
    def _():
        s_ref[...] = jnp.zeros((8, D), jnp.float32)

    col = jnp.sum(h, axis=0, keepdims=True)
    s_ref[...] = s_ref[...] + jnp.broadcast_to(col, (8, D))


def _final(parts, dparts, bias8):
    return pl.pallas_call(
        _final_body,
        grid=(N // GB,),
        in_specs=[
            pl.BlockSpec((1, GB, D), lambda i: (0, i, 0)),
            pl.BlockSpec((1, GB, D), lambda i: (1, i, 0)),
            pl.BlockSpec((1, GB, 16), lambda i: (0, i, 0)),
            pl.BlockSpec((1, GB, 16), lambda i: (1, i, 0)),
            pl.BlockSpec((8, D), lambda i: (0, 0)),
        ],
        out_specs=[
            pl.BlockSpec((GB, D), lambda i: (i, 0)),
            pl.BlockSpec((8, D), lambda i: (0, 0)),
        ],
        out_shape=[
            jax.ShapeDtypeStruct((N, D), jnp.float32),
            jax.ShapeDtypeStruct((8, D), jnp.float32),
        ],
    )(parts, parts, dparts, dparts, bias8)


QN = NPAD // 4
TRASH = QN
QTPR = QN // 16
QZC = QTPR // E
DZC = TPR // E


def _sc_agg_body(xp_hbm, idx_hbm, xv_out,
                 idx_all, rows_v, xe_all, idx_adj, zero_v, xv_sh, sem,
                 sem2):
    c = lax.axis_index("c")
    s = lax.axis_index("s")
    w = c * 16 + s

    def initbuf(i, carry):
        for l in range(D // 16):
            zero_v[i, pl.ds(l * 16, 16)] = jnp.zeros((16,), jnp.float32)
        return carry

    lax.fori_loop(0, E, initbuf, 0)

    pltpu.sync_copy(idx_hbm.at[w], idx_all)
    sems = (sem, sem2)

    def fire(q, g):
        for m in range(4):
            pltpu.async_copy(xp_hbm.at[idx_all.at[q, 4 * g + m]],
                             rows_v.at[g % 2, m], sems[g % 2])

    def drain(q, g):
        for m in range(4):
            pltpu.make_async_copy(xp_hbm.at[idx_all.at[q, 4 * g + m]],
                                  rows_v.at[g % 2, m], sems[g % 2]).wait()

    fire(0, 0)

    def chunk_gather(q, carry):
        for g in range(4):
            if g < 3:
                fire(q, g + 1)
            else:
                fire(jnp.minimum(q + 1, CHUNKS - 1), 0)
            drain(q, g)
            buf = g % 2

            def rowsum(r, cc):
                for l in range(D // 16):
                    acc = rows_v[buf, 0, r, pl.ds(l * 16, 16)]
                    for m in range(1, 4):
                        acc = acc + rows_v[buf, m, r, pl.ds(l * 16, 16)]
                    if g == 0:
                        xe_all[q * E + r, pl.ds(l * 16, 16)] = acc
                    else:
                        xe_all[q * E + r, pl.ds(l * 16, 16)] = (
                            acc + xe_all[q * E + r, pl.ds(l * 16, 16)])
                return cc

            lax.fori_loop(0, E, rowsum, 0)
        return carry

    lax.fori_loop(0, CHUNKS, chunk_gather, 0)
    drain(CHUNKS - 1, 0)

    def quarter_body(quarter, carry0):
        base = quarter * QN

        zh = [
            pltpu.async_copy(zero_v, xv_sh.at[pl.ds(s * QTPR + kk * E, E), :],
                             sem)
            for kk in range(QZC)
        ]
        for h in zh:
            h.wait()
        plsc.subcore_barrier()

        def chunk_scatter(q, carry):
            for m in range(16):
                for e2 in range(E // 16):
                    v = idx_all[q, m, pl.ds(e2 * 16, 16)]
                    inr = (v >= base) & (v < base + QN)
                    idx_adj[m, pl.ds(e2 * 16, 16)] = jnp.where(
                        inr, v - base, TRASH + m)
            handles = [
                pltpu.async_copy(xe_all.at[pl.ds(q * E, E), :],
                                 xv_sh.at[idx_adj.at[m]], sem, add=True)
                for m in range(16)
            ]
            for h in handles:
                h.wait()
            return carry

        lax.fori_loop(0, CHUNKS, chunk_scatter, 0)
        plsc.subcore_barrier()

        pltpu.sync_copy(xv_sh.at[pl.ds(s * QTPR, QTPR), :],
                        xv_out.at[c, pl.ds(base + s * QTPR, QTPR), :])
        return carry0

    lax.fori_loop(0, 4, quarter_body, 0)


@functools.lru_cache(maxsize=None)
def _sc_agg_kernel():
    return functools.partial(
        pl.kernel,
        out_type=jax.ShapeDtypeStruct((2, NPAD, D), jnp.float32),
        mesh=plsc.VectorSubcoreMesh(core_axis_name="c", subcore_axis_name="s",
                                    num_cores=2, num_subcores=16),
        scratch_types=[
            pltpu.VMEM((CHUNKS, 16, E), jnp.int32),
            pltpu.VMEM((2, 4, E, D), jnp.float32),
            pltpu.VMEM((PER_W, D), jnp.float32),
            pltpu.VMEM((16, E), jnp.int32),
            pltpu.VMEM((E, D), jnp.float32),
            pltpu.VMEM_SHARED((QN + E, D), jnp.float32),
            pltpu.SemaphoreType.DMA,
            pltpu.SemaphoreType.DMA,
        ],
    )(_sc_agg_body)


def _sc_degree_body(idx_hbm, dv_out, idx_all, ones_v, zerod_v, dv_sh):
    c = lax.axis_index("c")
    s = lax.axis_index("s")
    w = c * 16 + s

    def initbuf(i, carry):
        zerod_v[i, :] = jnp.zeros((16,), jnp.float32)
        ones_v[i, :] = jnp.ones((16,), jnp.float32)
        return carry

    lax.fori_loop(0, E, initbuf, 0)
    pltpu.sync_copy(idx_hbm.at[w], idx_all)

    def dzloop(kk, carry):
        pltpu.sync_copy(zerod_v, dv_sh.at[pl.ds(s * TPR + kk * E, E), :])
        return carry

    lax.fori_loop(0, DZC, dzloop, 0)
    plsc.subcore_barrier()

    def chunk_ones(q, carry):
        for m in range(16):
            pltpu.sync_copy(ones_v, dv_sh.at[idx_all.at[q, m]], add=True)
        return carry

    lax.fori_loop(0, CHUNKS, chunk_ones, 0)
    plsc.subcore_barrier()

    pltpu.sync_copy(dv_sh.at[pl.ds(s * TPR, TPR), :],
                    dv_out.at[c, pl.ds(s * TPR, TPR), :])


@functools.lru_cache(maxsize=None)
def _sc_degree_kernel():
    return functools.partial(
        pl.kernel,
        out_type=jax.ShapeDtypeStruct((2, NPAD, 16), jnp.float32),
        mesh=plsc.VectorSubcoreMesh(core_axis_name="c", subcore_axis_name="s",
                                    num_cores=2, num_subcores=16),
        scratch_types=[
            pltpu.VMEM((CHUNKS, 16, E), jnp.int32),
            pltpu.VMEM((E, 16), jnp.float32),
            pltpu.VMEM((E, 16), jnp.float32),
            pltpu.VMEM_SHARED((NPAD, 16), jnp.float32),
        ],
    )(_sc_degree_body)


def _sc_agg(xp, idxarr):
    return _sc_agg_kernel()(xp, idxarr)


def _sc_degree(idxarr):
    return _sc_degree_kernel()(idxarr)


def kernel(x, theta0, bias0, theta1, bias1, theta2, bias2, theta3, bias3,
           Wm, bm, Wa, ba):
    f32 = jnp.float32
    x = x.astype(f32)
    xpad = jnp.zeros((NPAD, D), f32).at[:N].set(x)
    nn_full = _topk(xpad, xpad.T)
    nnp = jnp.concatenate(
        [nn_full[:N], jnp.full((NPAD - N, K), DUMMY, jnp.int32)], axis=0)
    idxarr = nnp.T.reshape(16, NW, CHUNKS, E).transpose(1, 2, 0, 3)

    thetas = [theta1, theta2, theta3]
    biases = [bias0, bias1, bias2, bias3]

    dparts = _sc_degree(idxarr)
    h = _mm0(x, theta0)
    for L in range(3):
        parts = _sc_agg(h, idxarr)
        b8 = jnp.broadcast_to(biases[L][None, :], (8, D))
        h = _layer(parts, dparts, b8, thetas[L])
    parts = _sc_agg(h, idxarr)
    b8 = jnp.broadcast_to(biases[3][None, :], (8, D))
    feats, psum = _final(parts, dparts, b8)

    feats_pool = psum[0:1, :] * f32(1.0 / N)
    mean = (feats_pool @ Wm.T + bm)[0]
    alpha = (feats_pool @ Wa.T + ba)[0]
    return (jax.nn.sigmoid(mean), jnp.log(jax.nn.sigmoid(alpha)),
            feats, feats_pool)
